# Initial kernel scaffold; baseline (speedup 1.0000x reference)
import jax
import jax.numpy as jnp
from jax.experimental import pallas as pl


def kernel(x):
    raise NotImplementedError("write your pallas kernel here")


# speedup vs baseline: 4.5100x; 4.5100x over previous
import functools
import math

import jax
import jax.numpy as jnp
from jax.experimental import pallas as pl

_NBINS = 9
_POOL = 8
_GW = 16
_H = 224


def _gkern_tiled():
    n = jnp.arange(_GW, dtype=jnp.float32)
    n = n - jnp.mean(n)
    n = n / (_GW // 2)
    w = jnp.exp(-0.5 * n * n)
    g2 = w[:, None] * w[None, :]
    g2 = g2 / jnp.sum(g2)
    return jnp.tile(g2, (_H // _GW, _H // _GW))


def _pool_mat():
    i = jnp.arange(_H)[:, None] // _POOL
    j = jnp.arange(_H // _POOL)[None, :]
    return (i == j).astype(jnp.float32)


def _hog_kernel(xp_ref, gk_ref, p_ref, out_ref):
    xp = xp_ref[0].astype(jnp.float32)
    gk = gk_ref[...]
    pmat = p_ref[...]

    v = xp[0:224, :] + 2.0 * xp[1:225, :] + xp[2:226, :]
    gx = v[:, 0:224] - v[:, 2:226]
    h = xp[:, 0:224] + 2.0 * xp[:, 1:225] + xp[:, 2:226]
    gy = h[0:224, :] - h[2:226, :]

    norm = jnp.sqrt(gx * gx + gy * gy)
    phase = jnp.arctan2(gx, gy) / math.pi * float(_NBINS)
    idx = jnp.mod(jnp.floor(phase), float(_NBINS))
    val = norm * gk

    pt = pmat.T
    dot = functools.partial(
        jax.lax.dot_general,
        dimension_numbers=(((1,), (0,)), ((), ())),
        preferred_element_type=jnp.float32,
        precision=jax.lax.Precision.HIGHEST,
    )

    pooled = []
    ssq = jnp.zeros((_H // _POOL, _H // _POOL), dtype=jnp.float32)
    for k in range(_NBINS):
        mk = jnp.where(idx == float(k), val, 0.0)
        pk = dot(dot(pt, mk), pmat)
        pooled.append(pk)
        ssq = ssq + pk * pk

    den = jnp.maximum(jnp.sqrt(ssq), 1e-12)
    for k in range(_NBINS):
        out_ref[0, k, :, :] = pooled[k] / den


@jax.jit
def kernel(x):
    b, c, hh, ww = x.shape
    xp = jnp.pad(x, ((0, 0), (0, 0), (1, 1), (1, 1)), mode="reflect")
    xp = xp.reshape(b * c, hh + 2, ww + 2).astype(jnp.bfloat16)
    gk = _gkern_tiled()
    pmat = _pool_mat()

    hp = hh // _POOL
    out = pl.pallas_call(
        _hog_kernel,
        grid=(b * c,),
        in_specs=[
            pl.BlockSpec((1, hh + 2, ww + 2), lambda i: (i, 0, 0)),
            pl.BlockSpec((hh, ww), lambda i: (0, 0)),
            pl.BlockSpec((hh, hp), lambda i: (0, 0)),
        ],
        out_specs=pl.BlockSpec((1, _NBINS, hp, hp), lambda i: (i, 0, 0, 0)),
        out_shape=jax.ShapeDtypeStruct((b * c, _NBINS, hp, hp), jnp.float32),
    )(xp, gk, pmat)

    out_h = hh // _GW
    us = hp // out_h
    out = out.reshape(b, c, _NBINS, out_h, us, out_h, us)
    out = out.transpose(0, 1, 2, 4, 6, 3, 5)
    return out.reshape(b, c * _NBINS * us * us, out_h, out_h)


# speedup vs baseline: 7.3117x; 1.6212x over previous
import functools
import math

import jax
import jax.numpy as jnp
from jax.experimental import pallas as pl
from jax.experimental.pallas import tpu as pltpu

_NBINS = 9
_POOL = 8
_GW = 16
_H = 224


def _gkern_tiled():
    n = jnp.arange(_GW, dtype=jnp.float32)
    n = n - jnp.mean(n)
    n = n / (_GW // 2)
    w = jnp.exp(-0.5 * n * n)
    g2 = w[:, None] * w[None, :]
    g2 = g2 / jnp.sum(g2)
    return jnp.tile(g2, (_H // _GW, _H // _GW))


def _pool_mat():
    i = jnp.arange(_H)[:, None] // _POOL
    j = jnp.arange(_H // _POOL)[None, :]
    return (i == j).astype(jnp.float32)


_G = 4


def _hog_kernel(xp_ref, gk_ref, p_ref, out_ref):
    gk = gk_ref[...]
    pmat = p_ref[...]
    pt = pmat.T
    hp = _H // _POOL

    dot = functools.partial(
        jax.lax.dot_general,
        dimension_numbers=(((1,), (0,)), ((), ())),
        preferred_element_type=jnp.float32,
    )

    for g in range(_G):
        xp = xp_ref[g].astype(jnp.float32)

        v = xp[0:224, :] + 2.0 * xp[1:225, :] + xp[2:226, :]
        gx = v[:, 0:224] - v[:, 2:226]
        h = xp[:, 0:224] + 2.0 * xp[:, 1:225] + xp[:, 2:226]
        gy = h[0:224, :] - h[2:226, :]

        norm = jnp.sqrt(gx * gx + gy * gy)
        phase = jnp.arctan2(gx, gy) / math.pi * float(_NBINS)
        idx = jnp.mod(jnp.floor(phase).astype(jnp.int32), _NBINS)
        val = norm * gk

        pooled = []
        ssq = jnp.zeros((hp, hp), dtype=jnp.float32)
        for k in range(_NBINS):
            mk = jnp.where(idx == k, val, 0.0)
            pk = dot(dot(pt, mk), pmat)
            pooled.append(pk)
            ssq = ssq + pk * pk

        den = jnp.maximum(jnp.sqrt(ssq), 1e-12)
        for k in range(_NBINS):
            out_ref[g, k, :, :] = pooled[k] / den


@jax.jit
def kernel(x):
    b, c, hh, ww = x.shape
    xp = jnp.pad(x, ((0, 0), (0, 0), (1, 1), (1, 1)), mode="reflect")
    xp = xp.reshape(b * c, hh + 2, ww + 2).astype(jnp.bfloat16)
    gk = _gkern_tiled()
    pmat = _pool_mat()

    hp = hh // _POOL
    out = pl.pallas_call(
        _hog_kernel,
        grid=(b * c // _G,),
        in_specs=[
            pl.BlockSpec((_G, hh + 2, ww + 2), lambda i: (i, 0, 0)),
            pl.BlockSpec((hh, ww), lambda i: (0, 0)),
            pl.BlockSpec((hh, hp), lambda i: (0, 0)),
        ],
        out_specs=pl.BlockSpec((_G, _NBINS, hp, hp), lambda i: (i, 0, 0, 0)),
        out_shape=jax.ShapeDtypeStruct((b * c, _NBINS, hp, hp), jnp.float32),
        compiler_params=pltpu.CompilerParams(
            dimension_semantics=("parallel",),
        ),
    )(xp, gk, pmat)

    out_h = hh // _GW
    us = hp // out_h
    out = out.reshape(b, c, _NBINS, out_h, us, out_h, us)
    out = out.transpose(0, 1, 2, 4, 6, 3, 5)
    return out.reshape(b, c * _NBINS * us * us, out_h, out_h)


# speedup vs baseline: 22.2292x; 3.0402x over previous
import functools
import math

import jax
import jax.numpy as jnp
from jax.experimental import pallas as pl
from jax.experimental.pallas import tpu as pltpu

_NBINS = 9
_POOL = 8
_GW = 16
_H = 224
_HP = _H // _POOL
_OH = _H // _GW
_G = 4


def _gkern_tiled():
    n = jnp.arange(_GW, dtype=jnp.float32)
    n = n - jnp.mean(n)
    n = n / (_GW // 2)
    w = jnp.exp(-0.5 * n * n)
    g2 = w[:, None] * w[None, :]
    g2 = g2 / jnp.sum(g2)
    return jnp.tile(g2, (_H // _GW, _H // _GW))


def _pool_left():
    u = jnp.arange(_HP)[:, None]
    r, i = u // _OH, u % _OH
    col = jnp.arange(_H)[None, :]
    return (col // _POOL == 2 * i + r).astype(jnp.float32)


def _pool_right():
    w = jnp.arange(_HP)[None, :]
    s, j = w // _OH, w % _OH
    row = jnp.arange(_H)[:, None]
    return (row // _POOL == 2 * j + s).astype(jnp.float32)


def _hog_kernel(x_ref, gk_ref, pl_ref, pr_ref, out_ref):
    gk = gk_ref[...]
    plm = pl_ref[...]
    prm = pr_ref[...]

    dot = functools.partial(
        jax.lax.dot_general,
        dimension_numbers=(((1,), (0,)), ((), ())),
        preferred_element_type=jnp.float32,
    )

    for g in range(_G):
        xr = x_ref[g].astype(jnp.bfloat16).astype(jnp.float32)
        xv = jnp.concatenate([xr[1:2, :], xr, xr[_H - 2:_H - 1, :]], axis=0)
        xp = jnp.concatenate(
            [xv[:, 1:2], xv, xv[:, _H - 2:_H - 1]], axis=1)

        v = xp[0:_H, :] + 2.0 * xp[1:_H + 1, :] + xp[2:_H + 2, :]
        gx = v[:, 0:_H] - v[:, 2:_H + 2]
        h = xp[:, 0:_H] + 2.0 * xp[:, 1:_H + 1] + xp[:, 2:_H + 2]
        gy = h[0:_H, :] - h[2:_H + 2, :]

        norm = jnp.sqrt(gx * gx + gy * gy)
        val = norm * gk

        sgn = jnp.where(gx < 0.0, -1.0, 1.0)
        wx = gy * sgn
        wy = gx * sgn
        idx = jnp.zeros(gx.shape, dtype=jnp.int32)
        for m in range(1, _NBINS):
            a = m * math.pi / _NBINS
            cm = jnp.float32(math.cos(a))
            sm = jnp.float32(math.sin(a))
            idx = idx + (wy * cm - wx * sm > 0.0).astype(jnp.int32)
        idx = jnp.where((gx == 0.0) & (gy < 0.0), 0, idx)

        pooled = []
        ssq = jnp.zeros((_HP, _HP), dtype=jnp.float32)
        for k in range(_NBINS):
            mk = jnp.where(idx == k, val, 0.0)
            qk = dot(dot(plm, mk), prm)
            pooled.append(qk)
            ssq = ssq + qk * qk

        den = jnp.maximum(jnp.sqrt(ssq), 1e-12)
        for k in range(_NBINS):
            t = pooled[k] / den
            out_ref[g, k, 0, :, :] = t[0:_OH, 0:_OH]
            out_ref[g, k, 1, :, :] = t[0:_OH, _OH:_HP]
            out_ref[g, k, 2, :, :] = t[_OH:_HP, 0:_OH]
            out_ref[g, k, 3, :, :] = t[_OH:_HP, _OH:_HP]


@jax.jit
def kernel(x):
    b, c, hh, ww = x.shape
    xf = x.reshape(b * c, hh, ww)
    gk = _gkern_tiled()
    plm = _pool_left()
    prm = _pool_right()

    out = pl.pallas_call(
        _hog_kernel,
        grid=(b * c // _G,),
        in_specs=[
            pl.BlockSpec((_G, hh, ww), lambda i: (i, 0, 0)),
            pl.BlockSpec((_H, _H), lambda i: (0, 0)),
            pl.BlockSpec((_HP, _H), lambda i: (0, 0)),
            pl.BlockSpec((_H, _HP), lambda i: (0, 0)),
        ],
        out_specs=pl.BlockSpec(
            (_G, _NBINS, 4, _OH, _OH), lambda i: (i, 0, 0, 0, 0)),
        out_shape=jax.ShapeDtypeStruct(
            (b * c, _NBINS, 4, _OH, _OH), jnp.float32),
        compiler_params=pltpu.CompilerParams(
            dimension_semantics=("parallel",),
        ),
    )(xf, gk, plm, prm)

    return out.reshape(b, c * _NBINS * 4, _OH, _OH)


# speedup vs baseline: 25.4948x; 1.1469x over previous
import functools
import math

import jax
import jax.numpy as jnp
from jax.experimental import pallas as pl
from jax.experimental.pallas import tpu as pltpu

_NBINS = 9
_POOL = 8
_GW = 16
_H = 224
_HP = _H // _POOL
_OH = _H // _GW
_G = 4


def _gkern_tiled():
    n = jnp.arange(_GW, dtype=jnp.float32)
    n = n - jnp.mean(n)
    n = n / (_GW // 2)
    w = jnp.exp(-0.5 * n * n)
    g2 = w[:, None] * w[None, :]
    g2 = g2 / jnp.sum(g2)
    return jnp.tile(g2, (_H // _GW, _H // _GW))


def _pool_left():
    u = jnp.arange(_HP)[:, None]
    r, i = u // _OH, u % _OH
    col = jnp.arange(_H)[None, :]
    return (col // _POOL == 2 * i + r).astype(jnp.float32)


def _pool_right():
    w = jnp.arange(_HP)[None, :]
    s, j = w // _OH, w % _OH
    row = jnp.arange(_H)[:, None]
    return (row // _POOL == 2 * j + s).astype(jnp.float32)


def _hog_kernel(x_ref, gk_ref, pl_ref, pr_ref, out_ref):
    gk = gk_ref[...]
    plm = pl_ref[...]
    prm = pr_ref[...]

    dot = functools.partial(
        jax.lax.dot_general,
        dimension_numbers=(((1,), (0,)), ((), ())),
        preferred_element_type=jnp.float32,
    )

    for g in range(_G):
        xr = x_ref[g].astype(jnp.bfloat16).astype(jnp.float32)
        xv = jnp.concatenate([xr[1:2, :], xr, xr[_H - 2:_H - 1, :]], axis=0)
        xp = jnp.concatenate(
            [xv[:, 1:2], xv, xv[:, _H - 2:_H - 1]], axis=1)

        v = xp[0:_H, :] + 2.0 * xp[1:_H + 1, :] + xp[2:_H + 2, :]
        gx = v[:, 0:_H] - v[:, 2:_H + 2]
        h = xp[:, 0:_H] + 2.0 * xp[:, 1:_H + 1] + xp[:, 2:_H + 2]
        gy = h[0:_H, :] - h[2:_H + 2, :]

        norm = jnp.sqrt(gx * gx + gy * gy)
        val = norm * gk

        t = gx / gy
        idx = jnp.where(t < 0.0, 4, -4)
        for m in range(1, _NBINS):
            tm = jnp.float32(math.tan(m * math.pi / _NBINS))
            idx = idx + (t >= tm).astype(jnp.int32)

        pooled = []
        ssq = jnp.zeros((_HP, _HP), dtype=jnp.float32)
        for k in range(_NBINS):
            mk = jnp.where(idx == k, val, 0.0)
            qk = dot(dot(plm, mk), prm)
            pooled.append(qk)
            ssq = ssq + qk * qk

        den = jnp.maximum(jnp.sqrt(ssq), 1e-12)
        for k in range(_NBINS):
            q = pooled[k] / den
            out_ref[g, k, 0, :, :] = q[0:_OH, 0:_OH]
            out_ref[g, k, 1, :, :] = q[0:_OH, _OH:_HP]
            out_ref[g, k, 2, :, :] = q[_OH:_HP, 0:_OH]
            out_ref[g, k, 3, :, :] = q[_OH:_HP, _OH:_HP]


@jax.jit
def kernel(x):
    b, c, hh, ww = x.shape
    xf = x.reshape(b * c, hh, ww)
    gk = _gkern_tiled()
    plm = _pool_left()
    prm = _pool_right()

    out = pl.pallas_call(
        _hog_kernel,
        grid=(b * c // _G,),
        in_specs=[
            pl.BlockSpec((_G, hh, ww), lambda i: (i, 0, 0)),
            pl.BlockSpec((_H, _H), lambda i: (0, 0)),
            pl.BlockSpec((_HP, _H), lambda i: (0, 0)),
            pl.BlockSpec((_H, _HP), lambda i: (0, 0)),
        ],
        out_specs=pl.BlockSpec(
            (_G, _NBINS, 4, _OH, _OH), lambda i: (i, 0, 0, 0, 0)),
        out_shape=jax.ShapeDtypeStruct(
            (b * c, _NBINS, 4, _OH, _OH), jnp.float32),
        compiler_params=pltpu.CompilerParams(
            dimension_semantics=("parallel",),
        ),
    )(xf, gk, plm, prm)

    return out.reshape(b, c * _NBINS * 4, _OH, _OH)


# speedup vs baseline: 26.0996x; 1.0237x over previous
import functools
import math

import jax
import jax.numpy as jnp
from jax.experimental import pallas as pl
from jax.experimental.pallas import tpu as pltpu

_NBINS = 9
_POOL = 8
_GW = 16
_H = 224
_HP = _H // _POOL
_OH = _H // _GW
_G = 8


def _gkern_tiled():
    n = jnp.arange(_GW, dtype=jnp.float32)
    n = n - jnp.mean(n)
    n = n / (_GW // 2)
    w = jnp.exp(-0.5 * n * n)
    g2 = w[:, None] * w[None, :]
    g2 = g2 / jnp.sum(g2)
    return jnp.tile(g2, (_H // _GW, _H // _GW))


def _pool_left():
    u = jnp.arange(_HP)[:, None]
    r, i = u // _OH, u % _OH
    col = jnp.arange(_H)[None, :]
    return (col // _POOL == 2 * i + r).astype(jnp.float32)


def _pool_right():
    w = jnp.arange(_HP)[None, :]
    s, j = w // _OH, w % _OH
    row = jnp.arange(_H)[:, None]
    return (row // _POOL == 2 * j + s).astype(jnp.float32)


def _hog_kernel(x_ref, gk_ref, pl_ref, pr_ref, out_ref):
    gk = gk_ref[...]
    plm = pl_ref[...]
    prm = pr_ref[...]

    dot = functools.partial(
        jax.lax.dot_general,
        dimension_numbers=(((1,), (0,)), ((), ())),
        preferred_element_type=jnp.float32,
    )

    for g in range(_G):
        xr = x_ref[g].astype(jnp.bfloat16).astype(jnp.float32)
        xv = jnp.concatenate([xr[1:2, :], xr, xr[_H - 2:_H - 1, :]], axis=0)
        xp = jnp.concatenate(
            [xv[:, 1:2], xv, xv[:, _H - 2:_H - 1]], axis=1)

        v = xp[0:_H, :] + 2.0 * xp[1:_H + 1, :] + xp[2:_H + 2, :]
        gx = v[:, 0:_H] - v[:, 2:_H + 2]
        h = xp[:, 0:_H] + 2.0 * xp[:, 1:_H + 1] + xp[:, 2:_H + 2]
        gy = h[0:_H, :] - h[2:_H + 2, :]

        norm = jnp.sqrt(gx * gx + gy * gy)
        val = norm * gk

        t = gx / gy
        idx = jnp.where(t < 0.0, 4, -4)
        for m in range(1, _NBINS):
            tm = jnp.float32(math.tan(m * math.pi / _NBINS))
            idx = idx + (t >= tm).astype(jnp.int32)

        pooled = []
        ssq = jnp.zeros((_HP, _HP), dtype=jnp.float32)
        for k in range(_NBINS):
            mk = jnp.where(idx == k, val, 0.0)
            qk = dot(dot(plm, mk), prm)
            pooled.append(qk)
            ssq = ssq + qk * qk

        den = jnp.maximum(jnp.sqrt(ssq), 1e-12)
        for k in range(_NBINS):
            q = pooled[k] / den
            out_ref[g, k, 0, :, :] = q[0:_OH, 0:_OH]
            out_ref[g, k, 1, :, :] = q[0:_OH, _OH:_HP]
            out_ref[g, k, 2, :, :] = q[_OH:_HP, 0:_OH]
            out_ref[g, k, 3, :, :] = q[_OH:_HP, _OH:_HP]


@jax.jit
def kernel(x):
    b, c, hh, ww = x.shape
    xf = x.reshape(b * c, hh, ww)
    gk = _gkern_tiled()
    plm = _pool_left()
    prm = _pool_right()

    out = pl.pallas_call(
        _hog_kernel,
        grid=(b * c // _G,),
        in_specs=[
            pl.BlockSpec((_G, hh, ww), lambda i: (i, 0, 0)),
            pl.BlockSpec((_H, _H), lambda i: (0, 0)),
            pl.BlockSpec((_HP, _H), lambda i: (0, 0)),
            pl.BlockSpec((_H, _HP), lambda i: (0, 0)),
        ],
        out_specs=pl.BlockSpec(
            (_G, _NBINS, 4, _OH, _OH), lambda i: (i, 0, 0, 0, 0)),
        out_shape=jax.ShapeDtypeStruct(
            (b * c, _NBINS, 4, _OH, _OH), jnp.float32),
        compiler_params=pltpu.CompilerParams(
            dimension_semantics=("parallel",),
        ),
    )(xf, gk, plm, prm)

    return out.reshape(b, c * _NBINS * 4, _OH, _OH)


# speedup vs baseline: 26.8270x; 1.0279x over previous
import functools
import math

import jax
import jax.numpy as jnp
import numpy as np
from jax.experimental import pallas as pl
from jax.experimental.pallas import tpu as pltpu

_NBINS = 9
_POOL = 8
_GW = 16
_H = 224
_HP = _H // _POOL
_OH = _H // _GW
_G = 8


def _gkern_tiled():
    n = jnp.arange(_GW, dtype=jnp.float32)
    n = n - jnp.mean(n)
    n = n / (_GW // 2)
    w = jnp.exp(-0.5 * n * n)
    g2 = w[:, None] * w[None, :]
    g2 = g2 / jnp.sum(g2)
    return jnp.tile(g2, (_H // _GW, _H // _GW))


def _pool_left():
    u = np.arange(_HP)[:, None]
    r, i = u // _OH, u % _OH
    col = np.arange(_H)[None, :]
    return jnp.asarray((col // _POOL == 2 * i + r).astype(np.float32))


def _pool_right():
    w = np.arange(_HP)[None, :]
    s, j = w // _OH, w % _OH
    row = np.arange(_H)[:, None]
    return jnp.asarray((row // _POOL == 2 * j + s).astype(np.float32))


def _hog_kernel(x_ref, gk_ref, pl_ref, pr_ref, out_ref):
    gk = gk_ref[...]
    plm = pl_ref[...]
    prm = pr_ref[...]

    dot = functools.partial(
        jax.lax.dot_general,
        dimension_numbers=(((1,), (0,)), ((), ())),
        preferred_element_type=jnp.float32,
    )

    for g in range(_G):
        xr = x_ref[g].astype(jnp.bfloat16).astype(jnp.float32)
        xv = jnp.concatenate([xr[1:2, :], xr, xr[_H - 2:_H - 1, :]], axis=0)
        xp = jnp.concatenate(
            [xv[:, 1:2], xv, xv[:, _H - 2:_H - 1]], axis=1)

        v = xp[0:_H, :] + 2.0 * xp[1:_H + 1, :] + xp[2:_H + 2, :]
        gx = v[:, 0:_H] - v[:, 2:_H + 2]
        h = xp[:, 0:_H] + 2.0 * xp[:, 1:_H + 1] + xp[:, 2:_H + 2]
        gy = h[0:_H, :] - h[2:_H + 2, :]

        norm = jnp.sqrt(gx * gx + gy * gy)
        val = norm * gk

        t = gx / gy
        idx = jnp.where(t < 0.0, 4, -4)
        for m in range(1, _NBINS):
            tm = jnp.float32(math.tan(m * math.pi / _NBINS))
            idx = idx + (t >= tm).astype(jnp.int32)

        pooled = []
        ssq = jnp.zeros((_HP, _HP), dtype=jnp.float32)
        for k in range(_NBINS):
            mk = jnp.where(idx == k, val, 0.0)
            qk = dot(dot(plm, mk), prm)
            pooled.append(qk)
            ssq = ssq + qk * qk

        den = jnp.maximum(jnp.sqrt(ssq), 1e-12)
        for k in range(_NBINS):
            q = pooled[k] / den
            out_ref[g, k, 0, :, :] = q[0:_OH, 0:_OH]
            out_ref[g, k, 1, :, :] = q[0:_OH, _OH:_HP]
            out_ref[g, k, 2, :, :] = q[_OH:_HP, 0:_OH]
            out_ref[g, k, 3, :, :] = q[_OH:_HP, _OH:_HP]


@jax.jit
def kernel(x):
    b, c, hh, ww = x.shape
    xf = x.reshape(b * c, hh, ww)
    gk = _gkern_tiled()
    plm = _pool_left()
    prm = _pool_right()

    out = pl.pallas_call(
        _hog_kernel,
        grid=(b * c // _G,),
        in_specs=[
            pl.BlockSpec((_G, hh, ww), lambda i: (i, 0, 0)),
            pl.BlockSpec((_H, _H), lambda i: (0, 0)),
            pl.BlockSpec((_HP, _H), lambda i: (0, 0)),
            pl.BlockSpec((_H, _HP), lambda i: (0, 0)),
        ],
        out_specs=pl.BlockSpec(
            (_G, _NBINS, 4, _OH, _OH), lambda i: (i, 0, 0, 0, 0)),
        out_shape=jax.ShapeDtypeStruct(
            (b * c, _NBINS, 4, _OH, _OH), jnp.float32),
        compiler_params=pltpu.CompilerParams(
            dimension_semantics=("parallel",),
        ),
    )(xf, gk, plm, prm)

    return out.reshape(b, c * _NBINS * 4, _OH, _OH)
